# baseline (speedup 1.0000x reference)
import functools

import jax
import jax.numpy as jnp
from jax import lax
from jax.experimental import pallas as pl
from jax.experimental.pallas import tpu as pltpu
from jax.experimental.pallas import tpu_sc as plsc

DIM = 1024
NUM_CODES = 8192
KC = 8

TB = 256
CB = 4096

NEG_INF = float("-inf")
BIG_IDF = float(2**24)


def _extract_top8(vals, ids_f):
    out_v, out_i = [], []
    work = vals
    for _ in range(KC):
        m = jnp.max(work, axis=1, keepdims=True)
        hit = work == m
        sel = jnp.min(jnp.where(hit, ids_f, BIG_IDF), axis=1, keepdims=True)
        out_v.append(m)
        out_i.append(sel)
        work = jnp.where(ids_f == sel, NEG_INF, work)
    return jnp.concatenate(out_v, axis=1), jnp.concatenate(out_i, axis=1)


def _cmp_take_a(va, ia, vb, ib):
    return (va > vb) | ((va == vb) & (ia < ib))


def _rev8(x):
    return jnp.concatenate([x[:, i:i + 1] for i in range(KC - 1, -1, -1)],
                           axis=1)


def _swap_blocks(x, d):
    parts = []
    for i in range(0, KC, 2 * d):
        parts.append(x[:, i + d:i + 2 * d])
        parts.append(x[:, i:i + d])
    return jnp.concatenate(parts, axis=1)


def _merge8(tv, ti, bv, bi):
    rbv = _rev8(bv)
    rbi = _rev8(bi)
    ta = _cmp_take_a(tv, ti, rbv, rbi)
    hv = jnp.where(ta, tv, rbv)
    hi = jnp.where(ta, ti, rbi)
    for d in (4, 2, 1):
        pv = _swap_blocks(hv, d)
        pi = _swap_blocks(hi, d)
        ta = _cmp_take_a(hv, hi, pv, pi)
        keep_max = (lax.broadcasted_iota(jnp.int32, (TB, KC), 1) & d) == 0
        take_self = ta == keep_max
        hv = jnp.where(take_self, hv, pv)
        hi = jnp.where(take_self, hi, pi)
    return hv, hi


def _c2_body(c_ref, c2_ref):
    cbk = c_ref[...]
    c2_ref[...] = jnp.sum(cbk * cbk, axis=1)


def _code_norms(codebook):
    blk = 4096
    return pl.pallas_call(
        _c2_body,
        grid=(NUM_CODES // blk,),
        in_specs=[pl.BlockSpec((blk, DIM), lambda i: (i, 0))],
        out_specs=pl.BlockSpec((blk,), lambda i: (i,)),
        out_shape=jax.ShapeDtypeStruct((NUM_CODES,), jnp.float32),
    )(codebook)


def _topk_body(x_ref, c_ref, c2_ref, ids_ref, tv, ti):
    cb = pl.program_id(1)
    ncb = pl.num_programs(1)

    @pl.when(cb == 0)
    def _():
        tv[...] = jnp.full((TB, KC), NEG_INF, jnp.float32)
        ti[...] = jnp.zeros((TB, KC), jnp.float32)

    xb = x_ref[...]
    cbk = c_ref[...]
    xc = lax.dot_general(xb, cbk, (((1,), (1,)), ((), ())),
                         preferred_element_type=jnp.float32,
                         precision=lax.Precision.DEFAULT)
    c2 = c2_ref[...]
    x2 = jnp.sum(xb * xb, axis=1, keepdims=True)
    s = -((x2 + c2[None, :]) - 2.0 * xc)

    ids_f = (lax.broadcasted_iota(jnp.int32, (TB, CB), 1).astype(jnp.float32)
             + lax.convert_element_type(cb * CB, jnp.float32))
    bv, bi = _extract_top8(s, ids_f)

    nv, ni = _merge8(tv[...], ti[...], bv, bi)
    tv[...] = nv
    ti[...] = ni

    @pl.when(cb == ncb - 1)
    def _():
        ids_ref[...] = ni.astype(jnp.int32)


def _topk_ids(x2d, codebook, c2row):
    nt = x2d.shape[0]
    return pl.pallas_call(
        _topk_body,
        grid=(nt // TB, NUM_CODES // CB),
        in_specs=[
            pl.BlockSpec((TB, DIM), lambda tb, cb: (tb, 0)),
            pl.BlockSpec((CB, DIM), lambda tb, cb: (cb, 0)),
            pl.BlockSpec((CB,), lambda tb, cb: (cb,)),
        ],
        out_specs=pl.BlockSpec((TB, KC), lambda tb, cb: (tb, 0)),
        out_shape=jax.ShapeDtypeStruct((nt, KC), jnp.int32),
        scratch_shapes=[
            pltpu.VMEM((TB, KC), jnp.float32),
            pltpu.VMEM((TB, KC), jnp.float32),
        ],
        compiler_params=pltpu.CompilerParams(
            dimension_semantics=("parallel", "arbitrary"),
        ),
    )(x2d, codebook, c2row)


CT = 8


def _gather_avg(codebook, ids_flat, nt):
    info = plsc.get_sparse_core_info()
    nw = info.num_cores * info.num_subcores
    tpw = nt // nw
    nchunks = tpw // CT

    mesh = plsc.VectorSubcoreMesh(core_axis_name="c", subcore_axis_name="s")

    @functools.partial(
        pl.kernel,
        out_type=jax.ShapeDtypeStruct((nt, DIM), jnp.float32),
        mesh=mesh,
        scratch_types=[
            pltpu.VMEM((CT * KC,), jnp.int32),
            pltpu.VMEM((CT * KC, DIM), jnp.float32),
            pltpu.VMEM((CT, DIM), jnp.float32),
            pltpu.SemaphoreType.DMA,
        ],
    )
    def gather_kernel(cb_hbm, ids_hbm, out_hbm, idx_v, rows_v, out_v, sem):
        wid = lax.axis_index("s") * info.num_cores + lax.axis_index("c")
        tok0 = wid * tpw

        def chunk_body(ci, _):
            base = tok0 + ci * CT
            pltpu.sync_copy(ids_hbm.at[pl.ds(base * KC, CT * KC)], idx_v)
            pltpu.async_copy(cb_hbm.at[idx_v], rows_v, sem).wait()

            def col_body(c, _):
                for t in range(CT):
                    acc = rows_v[t * KC, pl.ds(c * 16, 16)]
                    for r in range(1, KC):
                        acc = acc + rows_v[t * KC + r, pl.ds(c * 16, 16)]
                    out_v[t, pl.ds(c * 16, 16)] = acc * 0.125
                return ()

            lax.fori_loop(0, DIM // 16, col_body, (), unroll=False)
            pltpu.sync_copy(out_v, out_hbm.at[pl.ds(base, CT)])
            return ()

        lax.fori_loop(0, nchunks, chunk_body, (), unroll=False)

    return gather_kernel(codebook, ids_flat)


def kernel(x, codebook):
    b, s, d = x.shape
    nt = b * s
    x2d = x.reshape(nt, d)
    ids = _topk_ids(x2d, codebook, _code_norms(codebook))
    out = _gather_avg(codebook, ids.reshape(nt * KC), nt)
    return out.reshape(b, s, d), ids.reshape(b, s, KC)


# speedup vs baseline: 1.0264x; 1.0264x over previous
import functools

import jax
import jax.numpy as jnp
from jax import lax
from jax.experimental import pallas as pl
from jax.experimental.pallas import tpu as pltpu
from jax.experimental.pallas import tpu_sc as plsc

DIM = 1024
NUM_CODES = 8192
KC = 8

TB = 256
CB = 4096

NEG_INF = float("-inf")
BIG_IDF = float(2**24)


def _extract_top8(vals, ids_f):
    out_v, out_i = [], []
    work = vals
    for _ in range(KC):
        m = jnp.max(work, axis=1, keepdims=True)
        hit = work == m
        sel = jnp.min(jnp.where(hit, ids_f, BIG_IDF), axis=1, keepdims=True)
        out_v.append(m)
        out_i.append(sel)
        work = jnp.where(ids_f == sel, NEG_INF, work)
    return jnp.concatenate(out_v, axis=1), jnp.concatenate(out_i, axis=1)


def _cmp_take_a(va, ia, vb, ib):
    return (va > vb) | ((va == vb) & (ia < ib))


def _rev8(x):
    return jnp.concatenate([x[:, i:i + 1] for i in range(KC - 1, -1, -1)],
                           axis=1)


def _swap_blocks(x, d):
    parts = []
    for i in range(0, KC, 2 * d):
        parts.append(x[:, i + d:i + 2 * d])
        parts.append(x[:, i:i + d])
    return jnp.concatenate(parts, axis=1)


def _merge8(tv, ti, bv, bi):
    rbv = _rev8(bv)
    rbi = _rev8(bi)
    ta = _cmp_take_a(tv, ti, rbv, rbi)
    hv = jnp.where(ta, tv, rbv)
    hi = jnp.where(ta, ti, rbi)
    for d in (4, 2, 1):
        pv = _swap_blocks(hv, d)
        pi = _swap_blocks(hi, d)
        ta = _cmp_take_a(hv, hi, pv, pi)
        keep_max = (lax.broadcasted_iota(jnp.int32, (TB, KC), 1) & d) == 0
        take_self = ta == keep_max
        hv = jnp.where(take_self, hv, pv)
        hi = jnp.where(take_self, hi, pi)
    return hv, hi


def _c2_body(c_ref, c2_ref):
    cbk = c_ref[...]
    c2_ref[...] = jnp.sum(cbk * cbk, axis=1)


def _code_norms(codebook):
    blk = 4096
    return pl.pallas_call(
        _c2_body,
        grid=(NUM_CODES // blk,),
        in_specs=[pl.BlockSpec((blk, DIM), lambda i: (i, 0))],
        out_specs=pl.BlockSpec((blk,), lambda i: (i,)),
        out_shape=jax.ShapeDtypeStruct((NUM_CODES,), jnp.float32),
    )(codebook)


def _topk_body(x_ref, c_ref, c2_ref, ids_ref, tv, ti):
    cb = pl.program_id(1)
    ncb = pl.num_programs(1)

    @pl.when(cb == 0)
    def _():
        tv[...] = jnp.full((TB, KC), NEG_INF, jnp.float32)
        ti[...] = jnp.zeros((TB, KC), jnp.float32)

    xb = x_ref[...]
    cbk = c_ref[...]
    xc = lax.dot_general(xb, cbk, (((1,), (1,)), ((), ())),
                         preferred_element_type=jnp.float32,
                         precision=lax.Precision.DEFAULT)
    c2 = c2_ref[...]
    x2 = jnp.sum(xb * xb, axis=1, keepdims=True)
    s = -((x2 + c2[None, :]) - 2.0 * xc)

    ids_f = (lax.broadcasted_iota(jnp.int32, (TB, CB), 1).astype(jnp.float32)
             + lax.convert_element_type(cb * CB, jnp.float32))
    bv, bi = _extract_top8(s, ids_f)

    nv, ni = _merge8(tv[...], ti[...], bv, bi)
    tv[...] = nv
    ti[...] = ni

    @pl.when(cb == ncb - 1)
    def _():
        ids_ref[...] = ni.astype(jnp.int32)


def _topk_ids(x2d, codebook, c2row):
    nt = x2d.shape[0]
    return pl.pallas_call(
        _topk_body,
        grid=(nt // TB, NUM_CODES // CB),
        in_specs=[
            pl.BlockSpec((TB, DIM), lambda tb, cb: (tb, 0)),
            pl.BlockSpec((CB, DIM), lambda tb, cb: (cb, 0)),
            pl.BlockSpec((CB,), lambda tb, cb: (cb,)),
        ],
        out_specs=pl.BlockSpec((TB, KC), lambda tb, cb: (tb, 0)),
        out_shape=jax.ShapeDtypeStruct((nt, KC), jnp.int32),
        scratch_shapes=[
            pltpu.VMEM((TB, KC), jnp.float32),
            pltpu.VMEM((TB, KC), jnp.float32),
        ],
        compiler_params=pltpu.CompilerParams(
            dimension_semantics=("parallel", "arbitrary"),
        ),
    )(x2d, codebook, c2row)


CT = 8


def _gather_avg(codebook, ids_flat, nt):
    info = plsc.get_sparse_core_info()
    nw = info.num_cores * info.num_subcores
    tpw = nt // nw
    nchunks = tpw // CT

    mesh = plsc.VectorSubcoreMesh(core_axis_name="c", subcore_axis_name="s")

    @functools.partial(
        pl.kernel,
        out_type=jax.ShapeDtypeStruct((nt, DIM), jnp.float32),
        mesh=mesh,
        scratch_types=[
            pltpu.VMEM((CT * KC,), jnp.int32),
            pltpu.VMEM((CT * KC, DIM), jnp.float32),
            pltpu.VMEM((CT, DIM), jnp.float32),
            pltpu.SemaphoreType.DMA,
        ],
    )
    def gather_kernel(cb_hbm, ids_hbm, out_hbm, idx_v, rows_v, out_v, sem):
        wid = lax.axis_index("s") * info.num_cores + lax.axis_index("c")
        tok0 = wid * tpw

        def chunk_body(ci, _):
            base = tok0 + ci * CT
            pltpu.sync_copy(ids_hbm.at[pl.ds(base * KC, CT * KC)], idx_v)
            pltpu.async_copy(cb_hbm.at[idx_v], rows_v, sem).wait()

            def col_body(c, _):
                for t in range(CT):
                    acc = rows_v[t * KC, pl.ds(c * 16, 16)]
                    for r in range(1, KC):
                        acc = acc + rows_v[t * KC + r, pl.ds(c * 16, 16)]
                    out_v[t, pl.ds(c * 16, 16)] = acc * 0.125
                return ()

            lax.fori_loop(0, DIM // 16, col_body, (), unroll=False)
            pltpu.sync_copy(out_v, out_hbm.at[pl.ds(base, CT)])
            return ()

        lax.fori_loop(0, nchunks, chunk_body, (), unroll=False)

    return gather_kernel(codebook, ids_flat)


def kernel(x, codebook):
    b, s, d = x.shape
    nt = b * s
    x2d = x.reshape(nt, d)
    c2row = _code_norms(codebook)
    half = nt // 2
    outs, idss = [], []
    for h in range(2):
        xh = lax.slice_in_dim(x2d, h * half, (h + 1) * half, axis=0)
        ids = _topk_ids(xh, codebook, c2row)
        outs.append(_gather_avg(codebook, ids.reshape(half * KC), half))
        idss.append(ids)
    out = jnp.concatenate(outs, axis=0)
    ids = jnp.concatenate(idss, axis=0)
    return out.reshape(b, s, d), ids.reshape(b, s, KC)
